# baseline (speedup 1.0000x reference)
import functools

import jax
import jax.numpy as jnp
from jax import lax
from jax.experimental import pallas as pl
from jax.experimental.pallas import tpu as pltpu
from jax.experimental.pallas import tpu_sc as plsc

NUM_E = 8192
DIM = 256
BATCH, CHAN, HGT, WID = 16, 256, 32, 32
M = BATCH * HGT * WID

TILE_M = 512
TILE_N = 1024
M_TILES = M // TILE_M
N_TILES = NUM_E // TILE_N


def _argmin_body(z_ref, e_ref, idx_ref, mind_ref):
    z = z_ref[...]
    z2 = jnp.sum(z * z, axis=1, keepdims=True)
    run_min = None
    run_idx = None
    for j in range(N_TILES):
        e = e_ref[pl.ds(j * TILE_N, TILE_N), :]
        m = lax.dot_general(
            z, e, (((1,), (1,)), ((), ())),
            precision=lax.Precision.HIGHEST,
            preferred_element_type=jnp.float32,
        )
        e2 = jnp.sum(e * e, axis=1)[None, :]
        d = (z2 - 2.0 * m) + e2
        loc_min = jnp.min(d, axis=1, keepdims=True)
        cols = lax.broadcasted_iota(jnp.int32, d.shape, 1)
        loc_idx = (
            jnp.min(jnp.where(d == loc_min, cols, jnp.int32(NUM_E)), axis=1)
            + j * TILE_N
        )
        loc_min = loc_min[:, 0]
        if j == 0:
            run_min, run_idx = loc_min, loc_idx
        else:
            better = loc_min < run_min
            run_idx = jnp.where(better, loc_idx, run_idx)
            run_min = jnp.where(better, loc_min, run_min)
    idx_ref[0] = run_idx[None, :]
    mind_ref[0] = run_min[None, :]


_NC, _NS = 2, 16
_NW = _NC * _NS
_BPW = M // _NW
_CH = 128
_NCH = _BPW // _CH


def _sc_gather_body(table_hbm, idx_hbm, out_hbm, idx_v, rows_v, sem):
    wid = lax.axis_index("s") * _NC + lax.axis_index("c")
    for c in range(_NCH):
        base = wid * _BPW + c * _CH
        pltpu.sync_copy(idx_hbm.at[pl.ds(base, _CH)], idx_v)
        pltpu.async_copy(table_hbm.at[idx_v], rows_v, sem).wait()
        pltpu.sync_copy(rows_v, out_hbm.at[pl.ds(base, _CH)])


@functools.cache
def _sc_gather():
    return pl.kernel(
        _sc_gather_body,
        out_type=jax.ShapeDtypeStruct((M, DIM), jnp.float32),
        mesh=plsc.VectorSubcoreMesh(
            core_axis_name="c", subcore_axis_name="s",
            num_cores=_NC, num_subcores=_NS,
        ),
        scratch_types=[
            pltpu.VMEM((_CH,), jnp.int32),
            pltpu.VMEM((_CH, DIM), jnp.float32),
            pltpu.SemaphoreType.DMA,
        ],
    )


def kernel(z, embeddings):
    zt = jnp.transpose(z, (0, 2, 3, 1)).reshape(M, DIM)
    idx3, mind3 = pl.pallas_call(
        _argmin_body,
        grid=(M_TILES,),
        in_specs=[
            pl.BlockSpec((TILE_M, DIM), lambda i: (i, 0)),
            pl.BlockSpec((NUM_E, DIM), lambda i: (0, 0)),
        ],
        out_specs=[
            pl.BlockSpec((1, 1, TILE_M), lambda i: (i, 0, 0)),
            pl.BlockSpec((1, 1, TILE_M), lambda i: (i, 0, 0)),
        ],
        out_shape=[
            jax.ShapeDtypeStruct((M_TILES, 1, TILE_M), jnp.int32),
            jax.ShapeDtypeStruct((M_TILES, 1, TILE_M), jnp.float32),
        ],
    )(zt, embeddings)
    idx = idx3.reshape(M)
    quantized = _sc_gather()(embeddings, idx)
    out = quantized.reshape(BATCH, HGT, WID, CHAN).transpose(0, 3, 1, 2)
    loss = 1.25 * jnp.sum(mind3) / (M * DIM)
    return out, loss


# speedup vs baseline: 2.4436x; 2.4436x over previous
import functools

import jax
import jax.numpy as jnp
from jax import lax
from jax.experimental import pallas as pl
from jax.experimental.pallas import tpu as pltpu
from jax.experimental.pallas import tpu_sc as plsc

NUM_E = 8192
DIM = 256
BATCH, CHAN, HGT, WID = 16, 256, 32, 32
M = BATCH * HGT * WID

TILE_M = 512
TILE_N = 1024
M_TILES = M // TILE_M
N_TILES = NUM_E // TILE_N


def _argmin_body(z_ref, e_ref, idx_ref, mind_ref, e2_ref):
    i = pl.program_id(0)

    @pl.when(i == 0)
    def _():
        e = e_ref[...]
        e2_ref[...] = jnp.sum(e * e, axis=1)[None, :]

    z = z_ref[...]
    z2 = jnp.sum(z * z, axis=1)
    zm2 = -2.0 * z
    run_min = None
    run_idx = None
    for j in range(N_TILES):
        e = e_ref[pl.ds(j * TILE_N, TILE_N), :]
        m2 = lax.dot_general(
            zm2, e, (((1,), (1,)), ((), ())),
            preferred_element_type=jnp.float32,
        )
        d = m2 + e2_ref[0, pl.ds(j * TILE_N, TILE_N)][None, :]
        loc_min = jnp.min(d, axis=1, keepdims=True)
        cols = lax.broadcasted_iota(jnp.int32, d.shape, 1)
        loc_idx = (
            jnp.min(jnp.where(d == loc_min, cols, jnp.int32(NUM_E)), axis=1)
            + j * TILE_N
        )
        loc_min = loc_min[:, 0]
        if j == 0:
            run_min, run_idx = loc_min, loc_idx
        else:
            better = loc_min < run_min
            run_idx = jnp.where(better, loc_idx, run_idx)
            run_min = jnp.where(better, loc_min, run_min)
    idx_ref[0] = run_idx[None, :]
    mind_ref[0] = (run_min + z2)[None, :]


_NC, _NS = 2, 16
_NW = _NC * _NS
_BPW = M // _NW
_CH = 128
_NCH = _BPW // _CH


def _sc_gather_body(table_hbm, idx_hbm, out_hbm, idx_v, rows_v, sem):
    wid = lax.axis_index("s") * _NC + lax.axis_index("c")
    for c in range(_NCH):
        base = wid * _BPW + c * _CH
        pltpu.sync_copy(idx_hbm.at[pl.ds(base, _CH)], idx_v)
        pltpu.async_copy(table_hbm.at[idx_v], rows_v, sem).wait()
        pltpu.sync_copy(rows_v, out_hbm.at[pl.ds(base, _CH)])


@functools.cache
def _sc_gather():
    return pl.kernel(
        _sc_gather_body,
        out_type=jax.ShapeDtypeStruct((M, DIM), jnp.float32),
        mesh=plsc.VectorSubcoreMesh(
            core_axis_name="c", subcore_axis_name="s",
            num_cores=_NC, num_subcores=_NS,
        ),
        scratch_types=[
            pltpu.VMEM((_CH,), jnp.int32),
            pltpu.VMEM((_CH, DIM), jnp.float32),
            pltpu.SemaphoreType.DMA,
        ],
    )


def kernel(z, embeddings):
    zt = jnp.transpose(z, (0, 2, 3, 1)).reshape(M, DIM)
    idx3, mind3 = pl.pallas_call(
        _argmin_body,
        grid=(M_TILES,),
        in_specs=[
            pl.BlockSpec((TILE_M, DIM), lambda i: (i, 0)),
            pl.BlockSpec((NUM_E, DIM), lambda i: (0, 0)),
        ],
        out_specs=[
            pl.BlockSpec((1, 1, TILE_M), lambda i: (i, 0, 0)),
            pl.BlockSpec((1, 1, TILE_M), lambda i: (i, 0, 0)),
        ],
        out_shape=[
            jax.ShapeDtypeStruct((M_TILES, 1, TILE_M), jnp.int32),
            jax.ShapeDtypeStruct((M_TILES, 1, TILE_M), jnp.float32),
        ],
        scratch_shapes=[pltpu.VMEM((1, NUM_E), jnp.float32)],
    )(zt, embeddings)
    idx = idx3.reshape(M)
    quantized = _sc_gather()(embeddings, idx)
    out = quantized.reshape(BATCH, HGT, WID, CHAN).transpose(0, 3, 1, 2)
    loss = 1.25 * jnp.sum(mind3) / (M * DIM)
    return out, loss


# speedup vs baseline: 2.4520x; 1.0034x over previous
import functools

import jax
import jax.numpy as jnp
from jax import lax
from jax.experimental import pallas as pl
from jax.experimental.pallas import tpu as pltpu
from jax.experimental.pallas import tpu_sc as plsc

NUM_E = 8192
DIM = 256
BATCH, CHAN, HGT, WID = 16, 256, 32, 32
M = BATCH * HGT * WID

TILE_M = 512
TILE_N = 1024
M_TILES = M // TILE_M
N_TILES = NUM_E // TILE_N


def _argmin_body(z_ref, e_ref, idx_ref, mind_ref, e2_ref):
    i = pl.program_id(0)

    @pl.when(i == 0)
    def _():
        e = e_ref[...]
        e2_ref[...] = jnp.sum(e * e, axis=1)[None, :]

    z = z_ref[...]
    z2 = jnp.sum(z * z, axis=1, keepdims=True)
    zm2 = -2.0 * z
    run_min = None
    run_idx = None
    for j in range(N_TILES):
        e = e_ref[pl.ds(j * TILE_N, TILE_N), :]
        m2 = lax.dot_general(
            zm2, e, (((1,), (1,)), ((), ())),
            preferred_element_type=jnp.float32,
        )
        d = m2 + e2_ref[0, pl.ds(j * TILE_N, TILE_N)][None, :]
        loc_min = jnp.min(d, axis=1, keepdims=True)
        cols = lax.broadcasted_iota(jnp.int32, d.shape, 1)
        loc_idx = jnp.min(
            jnp.where(d == loc_min, cols, jnp.int32(NUM_E)),
            axis=1, keepdims=True,
        ) + j * TILE_N
        if j == 0:
            run_min, run_idx = loc_min, loc_idx
        else:
            better = loc_min < run_min
            run_idx = jnp.where(better, loc_idx, run_idx)
            run_min = jnp.where(better, loc_min, run_min)
    idx_ref[0] = run_idx[:, 0][None, :]
    mind_ref[0] = (run_min + z2)[:, 0][None, :]


_NC, _NS = 2, 16
_NW = _NC * _NS
_BPW = M // _NW
_CH = 128
_NCH = _BPW // _CH


def _sc_gather_body(table_hbm, idx_hbm, out_hbm, idx_v, rows_v, sem):
    wid = lax.axis_index("s") * _NC + lax.axis_index("c")
    for c in range(_NCH):
        base = wid * _BPW + c * _CH
        pltpu.sync_copy(idx_hbm.at[pl.ds(base, _CH)], idx_v)
        pltpu.async_copy(table_hbm.at[idx_v], rows_v, sem).wait()
        pltpu.sync_copy(rows_v, out_hbm.at[pl.ds(base, _CH)])


@functools.cache
def _sc_gather():
    return pl.kernel(
        _sc_gather_body,
        out_type=jax.ShapeDtypeStruct((M, DIM), jnp.float32),
        mesh=plsc.VectorSubcoreMesh(
            core_axis_name="c", subcore_axis_name="s",
            num_cores=_NC, num_subcores=_NS,
        ),
        scratch_types=[
            pltpu.VMEM((_CH,), jnp.int32),
            pltpu.VMEM((_CH, DIM), jnp.float32),
            pltpu.SemaphoreType.DMA,
        ],
    )


def kernel(z, embeddings):
    zt = jnp.transpose(z, (0, 2, 3, 1)).reshape(M, DIM)
    idx3, mind3 = pl.pallas_call(
        _argmin_body,
        grid=(M_TILES,),
        in_specs=[
            pl.BlockSpec((TILE_M, DIM), lambda i: (i, 0)),
            pl.BlockSpec((NUM_E, DIM), lambda i: (0, 0)),
        ],
        out_specs=[
            pl.BlockSpec((1, 1, TILE_M), lambda i: (i, 0, 0)),
            pl.BlockSpec((1, 1, TILE_M), lambda i: (i, 0, 0)),
        ],
        out_shape=[
            jax.ShapeDtypeStruct((M_TILES, 1, TILE_M), jnp.int32),
            jax.ShapeDtypeStruct((M_TILES, 1, TILE_M), jnp.float32),
        ],
        scratch_shapes=[pltpu.VMEM((1, NUM_E), jnp.float32)],
    )(zt, embeddings)
    idx = idx3.reshape(M)
    quantized = _sc_gather()(embeddings, idx)
    out = quantized.reshape(BATCH, HGT, WID, CHAN).transpose(0, 3, 1, 2)
    loss = 1.25 * jnp.sum(mind3) / (M * DIM)
    return out, loss
